# baseline (speedup 1.0000x reference)
import jax
import jax.numpy as jnp
from jax.experimental import pallas as pl
from jax.experimental.pallas import tpu as pltpu

N = 1000
K = 104 * 104
BK = 2704
NKB = 4
MASK_THR = 0.5
SIGMA = 2.0


def _precompute(sr, lr, sel01_scr, selt01_scr, oh_scr, sc_scr):
    iu = jax.lax.broadcasted_iota(jnp.int32, (N, N), 0)
    iv = jax.lax.broadcasted_iota(jnp.int32, (N, N), 1)
    diag = iu == iv

    sc = jnp.sum(jnp.where(diag, jnp.broadcast_to(sr, (N, N)), 0.0),
                 axis=1, keepdims=True)
    lc = jnp.sum(jnp.where(diag, jnp.broadcast_to(lr, (N, N)), 0),
                 axis=1, keepdims=True)

    order = (sc > sr) | ((sc == sr) & (iu < iv))
    ordt = (sr > sc) | ((sr == sc) & (iv < iu))
    lbl = lc == lr

    sel01_scr[...] = (order & lbl).astype(jnp.float32)
    selt01_scr[...] = (ordt & lbl).astype(jnp.float32)
    rank_col = jnp.sum(ordt.astype(jnp.float32), axis=1, keepdims=True)
    oh_scr[...] = (rank_col == iv.astype(jnp.float32)).astype(jnp.float32)
    sc_scr[...] = sc


def _nms_epilogue(g, sel01, selt01, oh, sc):
    iu = jax.lax.broadcasted_iota(jnp.int32, (N, N), 0)
    iv = jax.lax.broadcasted_iota(jnp.int32, (N, N), 1)
    diag = iu == iv

    gd = jnp.where(diag, g, 0.0)
    s_col = jnp.sum(gd, axis=1, keepdims=True)
    s_row = jnp.sum(gd, axis=0, keepdims=True)

    den = s_col + s_row - g
    pos = den > 0.0
    iou = jnp.where(pos, g, 0.0) / jnp.where(pos, den, 1.0)

    m = sel01 * iou
    mt = selt01 * iou

    c_row = jnp.max(m, axis=0, keepdims=True)
    logr = c_row * c_row - mt * mt
    d_col = jnp.exp(SIGMA * jnp.min(logr, axis=1, keepdims=True))

    val_col = sc * d_col
    return jnp.sum(oh * val_col, axis=0, keepdims=True)


def _fused_kernel(xt_ref, sr_ref, lr_ref, out_ref, g_scr,
                  sel01_scr, selt01_scr, oh_scr, sc_scr):
    kb = pl.program_id(0)
    x = xt_ref[...]
    b = (x > MASK_THR).astype(jnp.float8_e4m3fn)
    part = jax.lax.dot_general(
        b, b, (((0,), (0,)), ((), ())), preferred_element_type=jnp.float32)

    @pl.when(kb == 0)
    def _():
        g_scr[...] = part
        _precompute(sr_ref[...], lr_ref[...],
                    sel01_scr, selt01_scr, oh_scr, sc_scr)

    @pl.when(kb != 0)
    def _():
        g_scr[...] += part

    @pl.when(kb == NKB - 1)
    def _():
        out_ref[...] = _nms_epilogue(
            g_scr[...], sel01_scr[...], selt01_scr[...], oh_scr[...],
            sc_scr[...])


def kernel(seg_masks_soft, cate_scores, cate_labels):
    xt = seg_masks_soft.transpose(1, 2, 0).reshape(K, N)
    sr = cate_scores.reshape(1, N)
    lr = cate_labels.reshape(1, N)
    out = pl.pallas_call(
        _fused_kernel,
        grid=(NKB,),
        in_specs=[
            pl.BlockSpec((BK, N), lambda kb: (kb, 0)),
            pl.BlockSpec((1, N), lambda kb: (0, 0)),
            pl.BlockSpec((1, N), lambda kb: (0, 0)),
        ],
        out_specs=pl.BlockSpec((1, N), lambda kb: (0, 0)),
        out_shape=jax.ShapeDtypeStruct((1, N), jnp.float32),
        scratch_shapes=[pltpu.VMEM((N, N), jnp.float32),
                        pltpu.VMEM((N, N), jnp.float32),
                        pltpu.VMEM((N, N), jnp.float32),
                        pltpu.VMEM((N, N), jnp.float32),
                        pltpu.VMEM((N, 1), jnp.float32)],
    )(xt, sr, lr)
    return out.reshape(N)


# speedup vs baseline: 1.0199x; 1.0199x over previous
import jax
import jax.numpy as jnp
from jax.experimental import pallas as pl
from jax.experimental.pallas import tpu as pltpu

N = 1000
K = 104 * 104
BK = 2704
NKB = 4
MASK_THR = 0.5
SIGMA = 2.0


def _nms_epilogue(g, sr, lr):
    iu = jax.lax.broadcasted_iota(jnp.int32, (N, N), 0)
    iv = jax.lax.broadcasted_iota(jnp.int32, (N, N), 1)
    diag = iu == iv

    sc = jnp.sum(jnp.where(diag, jnp.broadcast_to(sr, (N, N)), 0.0),
                 axis=1, keepdims=True)
    lc = jnp.sum(jnp.where(diag, jnp.broadcast_to(lr, (N, N)), 0),
                 axis=1, keepdims=True)

    gd = jnp.where(diag, g, 0.0)
    s_col = jnp.sum(gd, axis=1, keepdims=True)
    s_row = jnp.sum(gd, axis=0, keepdims=True)

    den = s_col + s_row - g
    pos = den > 0.0
    iou = jnp.where(pos, g, 0.0) / jnp.where(pos, den, 1.0)

    order = (sc > sr) | ((sc == sr) & (iu < iv))
    ordt = (sr > sc) | ((sr == sc) & (iv < iu))
    lbl = lc == lr

    m = jnp.where(order & lbl, iou, 0.0)
    mt = jnp.where(ordt & lbl, iou, 0.0)

    c_row = jnp.max(m, axis=0, keepdims=True)
    logr = c_row * c_row - mt * mt
    d_col = jnp.exp(SIGMA * jnp.min(logr, axis=1, keepdims=True))

    val_col = sc * d_col
    rank_col = jnp.sum(ordt.astype(jnp.float32), axis=1, keepdims=True)
    onehot = rank_col == iv.astype(jnp.float32)
    return jnp.sum(jnp.where(onehot, val_col, 0.0),
                   axis=0, keepdims=True)


def _fused_kernel(xt_ref, sr_ref, lr_ref, out_ref, g_scr):
    kb = pl.program_id(0)
    x = xt_ref[...]
    b = (x > MASK_THR).astype(jnp.float8_e4m3fn)
    part = jax.lax.dot_general(
        b, b, (((0,), (0,)), ((), ())), preferred_element_type=jnp.float32)

    @pl.when(kb == 0)
    def _():
        g_scr[...] = part

    @pl.when(kb != 0)
    def _():
        g_scr[...] += part

    @pl.when(kb == NKB - 1)
    def _():
        out_ref[...] = _nms_epilogue(g_scr[...], sr_ref[...], lr_ref[...])


def kernel(seg_masks_soft, cate_scores, cate_labels):
    xt = seg_masks_soft.transpose(1, 2, 0).reshape(K, N)
    sr = cate_scores.reshape(1, N)
    lr = cate_labels.reshape(1, N)
    out = pl.pallas_call(
        _fused_kernel,
        grid=(NKB,),
        in_specs=[
            pl.BlockSpec((BK, N), lambda kb: (kb, 0)),
            pl.BlockSpec((1, N), lambda kb: (0, 0)),
            pl.BlockSpec((1, N), lambda kb: (0, 0)),
        ],
        out_specs=pl.BlockSpec((1, N), lambda kb: (0, 0)),
        out_shape=jax.ShapeDtypeStruct((1, N), jnp.float32),
        scratch_shapes=[pltpu.VMEM((N, N), jnp.float32)],
    )(xt, sr, lr)
    return out.reshape(N)
